# baseline (speedup 1.0000x reference)
import jax
import jax.numpy as jnp
from jax import lax
from jax.experimental import pallas as pl
from jax.experimental.pallas import tpu as pltpu
from jax.experimental.pallas import tpu_sc as plsc

MAX_AA = 22
K_EMB = 16

_B, _L, _NBR = 4, 2048, 48
_NW = 32
_TPB = _NW // _B
_KPW = _NBR // _TPB
_TBL_C = 512


def _body(aa_hbm, e_hbm, m_hbm, tblt_hbm, out_hbm,
          aa_v, tblt_v, ev, mv, ob, sin, sout):
    cid = lax.axis_index("c")
    sid = lax.axis_index("s")
    w = sid * 2 + cid
    b = w // _TPB
    k0 = (w % _TPB) * _KPW

    iota16 = lax.broadcasted_iota(jnp.int32, (16,), 0)
    zeros = jnp.zeros((16,), jnp.float32)
    z1 = 44 + 22 * iota16
    z2 = 396 + 22 * iota16
    zmask = iota16 < 4
    dvecs = [jnp.full((16,), d, jnp.int32) for d in range(K_EMB)]

    def mask_table():
        for d in range(K_EMB):
            plsc.store_scatter(tblt_v, [dvecs[d], z1], zeros)
            plsc.store_scatter(tblt_v, [dvecs[d], z2], zeros, mask=zmask)

    def stage_in(kk, p):
        k = k0 + kk
        kb, kr = k // 8, k % 8
        de = pltpu.async_copy(e_hbm.at[b, kb, :, kr, :], ev[p], sin[p])
        dm = pltpu.async_copy(m_hbm.at[b, kb, :, kr, :], mv[p], sin[p])
        return de, dm

    def slab_compute(p, po):
        @plsc.parallel_loop(0, _L // 16, step=1, unroll=6)
        def body_ig(ig):
            ib = ig // 8
            lo = (ig % 8) * 16
            so = pl.ds(lo, 16)
            s = pl.ds(ig * 16, 16)
            u22 = (aa_v[s] + 1) * MAX_AA
            v = plsc.load_gather(aa_v, [ev[p][ib, so]]) + 1
            idx = u22 + v
            mk = mv[p][ib, so]
            for d in range(K_EMB):
                col = plsc.load_gather(tblt_v, [dvecs[d], idx])
                ob[po][d // 8, ib, d % 8, so] = col * mk

    in_descs = [stage_in(0, 0), None]
    da = pltpu.async_copy(aa_hbm.at[b], aa_v, sout[2])
    dt = pltpu.async_copy(tblt_hbm, tblt_v, sout[2])
    da.wait()
    dt.wait()
    mask_table()
    out_descs = [None, None, None]
    for kk in range(_KPW):
        p = kk % 2
        po = kk % 3
        if kk + 1 < _KPW:
            in_descs[1 - p] = stage_in(kk + 1, 1 - p)
        for d in in_descs[p]:
            d.wait()
        if out_descs[po] is not None:
            out_descs[po].wait()
        slab_compute(p, po)
        out_descs[po] = pltpu.async_copy(ob[po], out_hbm.at[b, k0 + kk], sout[po])
    for d in out_descs:
        d.wait()


@jax.jit
def kernel(aa, E_idx, mask_attend, table):
    aa32 = aa.astype(jnp.int32)
    e5 = E_idx.astype(jnp.int32).reshape(_B, _L // 128, 128, _NBR // 8, 8)
    e5 = jnp.transpose(e5, (0, 3, 1, 4, 2))
    m5 = mask_attend.reshape(_B, _L // 128, 128, _NBR // 8, 8)
    m5 = jnp.transpose(m5, (0, 3, 1, 4, 2))
    tblt = jnp.pad(table.T, ((0, 0), (0, _TBL_C - MAX_AA * MAX_AA)))

    mesh = plsc.VectorSubcoreMesh(core_axis_name="c", subcore_axis_name="s")
    run = pl.kernel(
        _body,
        out_type=jax.ShapeDtypeStruct(
            (_B, _NBR, K_EMB // 8, _L // 128, 8, 128), jnp.float32),
        mesh=mesh,
        compiler_params=pltpu.CompilerParams(
            needs_layout_passes=False, use_tc_tiling_on_sc=False),
        scratch_types=[
            pltpu.VMEM((_L,), jnp.int32),
            pltpu.VMEM((K_EMB, _TBL_C), jnp.float32),
            [pltpu.VMEM((_L // 128, 128), jnp.int32)] * 2,
            [pltpu.VMEM((_L // 128, 128), jnp.float32)] * 2,
            [pltpu.VMEM((K_EMB // 8, _L // 128, 8, 128), jnp.float32)] * 3,
            [pltpu.SemaphoreType.DMA] * 2,
            [pltpu.SemaphoreType.DMA] * 3,
        ],
    )
    out = run(aa32, e5, m5, tblt)
    out = jnp.transpose(out, (0, 3, 5, 1, 2, 4))
    return out.reshape(_B, _L, _NBR, K_EMB)


# speedup vs baseline: 1.0318x; 1.0318x over previous
import jax
import jax.numpy as jnp
from jax import lax
from jax.experimental import pallas as pl
from jax.experimental.pallas import tpu as pltpu
from jax.experimental.pallas import tpu_sc as plsc

MAX_AA = 22
K_EMB = 16

_B, _L, _NBR = 4, 2048, 48
_NW = 32
_TPB = _NW // _B
_KPW = _NBR // _TPB
_TBL_C = 512


def _body(aa_hbm, e_hbm, m_hbm, tblt_hbm, out_hbm,
          aa_v, tblt_v, ev, mv, ob, sin, sout):
    cid = lax.axis_index("c")
    sid = lax.axis_index("s")
    w = sid * 2 + cid
    b = w // _TPB
    k0 = (w % _TPB) * _KPW

    iota16 = lax.broadcasted_iota(jnp.int32, (16,), 0)
    zeros = jnp.zeros((16,), jnp.float32)
    z1 = 44 + 22 * iota16
    z2 = 396 + 22 * iota16
    zmask = iota16 < 4
    dvecs = [jnp.full((16,), d, jnp.int32) for d in range(K_EMB)]

    def mask_table():
        for d in range(K_EMB):
            plsc.store_scatter(tblt_v, [dvecs[d], z1], zeros)
            plsc.store_scatter(tblt_v, [dvecs[d], z2], zeros, mask=zmask)

    def stage_in(kk, p):
        k = k0 + kk
        kb, kr = k // 8, k % 8
        de = pltpu.async_copy(e_hbm.at[b, kb, :, kr, :], ev[p], sin[p])
        dm = pltpu.async_copy(m_hbm.at[b, kb, :, kr, :], mv[p], sin[p])
        return de, dm

    def slab_compute(p, po):
        @plsc.parallel_loop(0, _L // 16, step=1, unroll=4)
        def body_ig(ig):
            ib = ig // 8
            lo = (ig % 8) * 16
            so = pl.ds(lo, 16)
            s = pl.ds(ig * 16, 16)
            u22 = (aa_v[s] + 1) * MAX_AA
            v = plsc.load_gather(aa_v, [ev[p][ib, so]]) + 1
            idx = u22 + v
            mk = mv[p][ib, so]
            for d in range(K_EMB):
                col = plsc.load_gather(tblt_v, [dvecs[d], idx])
                ob[po][d // 8, ib, d % 8, so] = col * mk

    in_descs = [stage_in(0, 0), None]
    da = pltpu.async_copy(aa_hbm.at[b], aa_v, sout[2])
    dt = pltpu.async_copy(tblt_hbm, tblt_v, sout[2])
    da.wait()
    dt.wait()
    mask_table()
    out_descs = [None, None, None]
    for kk in range(_KPW):
        p = kk % 2
        po = kk % 3
        if kk + 1 < _KPW:
            in_descs[1 - p] = stage_in(kk + 1, 1 - p)
        for d in in_descs[p]:
            d.wait()
        if out_descs[po] is not None:
            out_descs[po].wait()
        slab_compute(p, po)
        out_descs[po] = pltpu.async_copy(ob[po], out_hbm.at[b, k0 + kk], sout[po])
    for d in out_descs:
        d.wait()


@jax.jit
def kernel(aa, E_idx, mask_attend, table):
    aa32 = aa.astype(jnp.int32)
    e5 = E_idx.astype(jnp.int32).reshape(_B, _L // 128, 128, _NBR // 8, 8)
    e5 = jnp.transpose(e5, (0, 3, 1, 4, 2))
    m5 = mask_attend.reshape(_B, _L // 128, 128, _NBR // 8, 8)
    m5 = jnp.transpose(m5, (0, 3, 1, 4, 2))
    tblt = jnp.pad(table.T, ((0, 0), (0, _TBL_C - MAX_AA * MAX_AA)))

    mesh = plsc.VectorSubcoreMesh(core_axis_name="c", subcore_axis_name="s")
    run = pl.kernel(
        _body,
        out_type=jax.ShapeDtypeStruct(
            (_B, _NBR, K_EMB // 8, _L // 128, 8, 128), jnp.float32),
        mesh=mesh,
        compiler_params=pltpu.CompilerParams(
            needs_layout_passes=False, use_tc_tiling_on_sc=False),
        scratch_types=[
            pltpu.VMEM((_L,), jnp.int32),
            pltpu.VMEM((K_EMB, _TBL_C), jnp.float32),
            [pltpu.VMEM((_L // 128, 128), jnp.int32)] * 2,
            [pltpu.VMEM((_L // 128, 128), jnp.float32)] * 2,
            [pltpu.VMEM((K_EMB // 8, _L // 128, 8, 128), jnp.float32)] * 3,
            [pltpu.SemaphoreType.DMA] * 2,
            [pltpu.SemaphoreType.DMA] * 3,
        ],
    )
    out = run(aa32, e5, m5, tblt)
    out = jnp.transpose(out, (0, 3, 5, 1, 2, 4))
    return out.reshape(_B, _L, _NBR, K_EMB)
